# baseline (speedup 1.0000x reference)
import functools

import jax
import jax.numpy as jnp
from jax import lax
from jax.experimental import pallas as pl
from jax.experimental.pallas import tpu as pltpu
from jax.experimental.pallas import tpu_sc as plsc

_N = 256
_D = 256
_TOPK = 5
_L = 16
_H = _D // 2
_NBH = _H // _L
_PAD_IDX = 9999.0


def _splat_f(v):
    return jnp.full((_L,), v, jnp.float32)


def _splat_i(v):
    return jnp.full((_L,), v, jnp.int32)


def _tree_reduce(vals, op):
    vals = list(vals)
    while len(vals) > 1:
        nxt = [op(vals[i], vals[i + 1]) for i in range(0, len(vals) - 1, 2)]
        if len(vals) % 2:
            nxt.append(vals[-1])
        vals = nxt
    return vals[0]


def _sc_top5_body(feat_hbm, val_hbm, idx_hbm, rows_v, val_v, idx_v):
    cid = lax.axis_index("c")
    sid = lax.axis_index("s")
    rbase = sid * _L
    cbase = cid * _H
    pltpu.sync_copy(feat_hbm.at[pl.ds(rbase, _L), pl.ds(cbase, _H)], rows_v)
    lane = lax.iota(jnp.int32, _L)
    neg = _splat_f(-jnp.inf)
    cbase_v = lax.broadcast(cbase, (_L,))

    bmax = []
    for k in range(_NBH):
        cols = [plsc.load_gather(rows_v, [lane, _splat_i(k * _L + c)])
                for c in range(_L)]
        bmax.append(_tree_reduce(cols, jnp.maximum))

    for p in range(_TOPK):
        m = _tree_reduce(bmax, jnp.maximum)
        bidx = _tree_reduce(
            [jnp.where(bmax[k] == m, _splat_i(k), _splat_i(_NBH))
             for k in range(_NBH)], jnp.minimum)
        gcol = bidx * _L
        gs = [plsc.load_gather(rows_v, [lane, gcol + c]) for c in range(_L)]
        cidx = _tree_reduce(
            [jnp.where(gs[c] == m, _splat_i(c), _splat_i(_L))
             for c in range(_L)], jnp.minimum)
        lidx = gcol + cidx
        plsc.store_scatter(rows_v, [lane, lidx], neg)
        nb = _tree_reduce(
            [jnp.where(cidx == c, neg, gs[c]) for c in range(_L)],
            jnp.maximum)
        bmax = [jnp.where(bidx == k, nb, bmax[k]) for k in range(_NBH)]
        plsc.store_scatter(val_v, [lane, _splat_i(p)], m)
        plsc.store_scatter(idx_v, [lane, _splat_i(p)],
                           (lidx + cbase_v).astype(jnp.float32))

    for p in range(_TOPK, 8):
        plsc.store_scatter(val_v, [lane, _splat_i(p)], neg)
        plsc.store_scatter(idx_v, [lane, _splat_i(p)], _splat_f(_PAD_IDX))

    obase = cid * 8
    pltpu.sync_copy(val_v, val_hbm.at[pl.ds(rbase, _L), pl.ds(obase, 8)])
    pltpu.sync_copy(idx_v, idx_hbm.at[pl.ds(rbase, _L), pl.ds(obase, 8)])


@functools.cache
def _sc_top5():
    return pl.kernel(
        _sc_top5_body,
        out_type=(jax.ShapeDtypeStruct((_N, 16), jnp.float32),
                  jax.ShapeDtypeStruct((_N, 16), jnp.float32)),
        mesh=plsc.VectorSubcoreMesh(core_axis_name="c",
                                    subcore_axis_name="s"),
        scratch_types=[
            pltpu.VMEM((_L, _H), jnp.float32),
            pltpu.VMEM((_L, 8), jnp.float32),
            pltpu.VMEM((_L, 8), jnp.float32),
        ],
        compiler_params=pltpu.CompilerParams(use_tc_tiling_on_sc=False,
                                             needs_layout_passes=False),
    )


def _tc_loss_kernel(prob1_ref, prob2_ref, val_ref, idx_ref, out_ref):
    vals = val_ref[...]
    idxs = idx_ref[...]
    colf = jax.lax.broadcasted_iota(jnp.int32, (_N, _D), 1).astype(jnp.float32)
    mask = jnp.zeros((_N, _D), jnp.float32)
    for _ in range(_TOPK):
        m = jnp.max(vals, axis=1, keepdims=True)
        sel = jnp.min(jnp.where(vals == m, idxs, _PAD_IDX),
                      axis=1, keepdims=True)
        vals = jnp.where(idxs == sel, -jnp.inf, vals)
        mask = jnp.where(colf == sel, 1.0, mask)

    mask_bf = mask.astype(jnp.bfloat16)
    overlap = jax.lax.dot_general(
        mask_bf, mask_bf, (((1,), (1,)), ((), ())),
        preferred_element_type=jnp.float32)
    target = overlap > (_TOPK - 0.5)

    sim = jax.lax.dot_general(
        prob2_ref[...].astype(jnp.bfloat16),
        prob1_ref[...].astype(jnp.bfloat16),
        (((1,), (1,)), ((), ())),
        preferred_element_type=jnp.float32)
    eps = 1e-12
    p = jnp.clip(sim, eps, 1.0 - eps)
    q = jnp.where(target, p, 1.0 - p)
    out_ref[0, 0] = -jnp.sum(jnp.log(q)) / (_N * _N)


def kernel(feat1, feat2, prob1, prob2):
    del feat2
    cval, cidx = _sc_top5()(feat1)
    out = pl.pallas_call(
        _tc_loss_kernel,
        out_shape=jax.ShapeDtypeStruct((1, 1), jnp.float32),
        out_specs=pl.BlockSpec(memory_space=pltpu.SMEM),
    )(prob1, prob2, cval, cidx)
    return out.reshape(())
